# baseline (speedup 1.0000x reference)
import functools

import numpy as np
import jax
import jax.numpy as jnp
from jax import lax
from jax.experimental import pallas as pl
from jax.experimental.pallas import tpu as pltpu
from jax.experimental.pallas import tpu_sc as plsc

_L = 40
_B, _T, _C = 64, 1024, 43
_NEG = -1e30


def _topology(num_labels):
    s = ["0 0 1", "0 1 2", "1 1 2"]
    for i in range(num_labels):
        sym = 3 + i
        s.append("0 0 %d" % sym)
        s.append("1 0 %d" % sym)
    s.append("0 2 -1")
    s.sort()
    return [tuple(int(x) for x in line.split()) for line in s]


_ARCS = _topology(_L)
_A = len(_ARCS)
_SRCA = np.array([a[0] for a in _ARCS], dtype=np.int32)
_DSTA = np.array([a[1] for a in _ARCS], dtype=np.int32)
_LABA = np.array([a[2] for a in _ARCS], dtype=np.int32)

_PAD = 128
_MS0 = np.zeros((1, _PAD), np.float32)
_MS0[0, :_A] = (_SRCA == 0)
_MS1 = np.zeros((1, _PAD), np.float32)
_MS1[0, :_A] = (_SRCA == 1)
_M0 = np.zeros((_PAD, _PAD), np.float32)
_M1 = np.zeros((_PAD, _PAD), np.float32)
for _a in range(_A):
    if _LABA[_a] >= 0 and _DSTA[_a] == 0:
        if _SRCA[_a] == 0:
            _M0[_a, _LABA[_a]] = 1.0
        else:
            _M1[_a, _LABA[_a]] = 1.0
_OH01 = np.zeros((1, _PAD), np.float32)
_OH11 = np.zeros((1, _PAD), np.float32)
_OHF = np.zeros((1, _PAD), np.float32)
for _a in range(_A):
    if _LABA[_a] < 0:
        _OHF[0, _a] = 1.0
    elif _SRCA[_a] == 0 and _DSTA[_a] == 1:
        _OH01[0, _a] = 1.0
    elif _SRCA[_a] == 1 and _DSTA[_a] == 1:
        _OH11[0, _a] = 1.0
_OHC2 = np.zeros((1, _PAD), np.float32)
_OHC2[0, 2] = 1.0

_ONESC = np.zeros((1, _PAD), np.float32)
_ONESC[0, :_C] = 1.0

_CS = np.concatenate([_MS0, _MS1, _OH01, _OH11, _OHF, _OHC2, _ONESC,
                      np.zeros((1, _PAD), np.float32)], axis=0)
_M01 = np.concatenate([_M0, _M1], axis=0)

_GB = 8
_GRID = _B // _GB


def _lse2(x, y):
    m = jnp.maximum(x, y)
    return m + jnp.log1p(jnp.exp(-jnp.abs(x - y)))


def _den_body(cs_ref, m_ref, dens_ref, lens_ref, labs_ref, lp_ref,
              out_ref, g_ref):
    d = dens_ref[...]
    cs = cs_ref[...]
    ms0 = cs[0:1, :]
    ms1 = cs[1:2, :]
    e = jnp.exp(d)
    lse0 = jnp.log(jnp.sum(e * ms0))
    lse1 = jnp.log(jnp.sum(e * ms1))
    w = d - ms0 * lse0 - ms1 * lse1
    ew = jnp.exp(w) * (ms0 + ms1)
    u0 = jnp.dot(ew, m_ref[0:_PAD, :])
    u1 = jnp.dot(ew, m_ref[_PAD:, :])
    c01 = jnp.sum(w * cs[2:3, :])
    c11 = jnp.sum(w * cs[3:4, :])
    wf = jnp.sum(w * cs[4:5, :])

    lp = lp_ref[...]
    labs = labs_ref[...]
    ci = lax.broadcasted_iota(jnp.int32, (_GB, _T, _C), 2)
    mlp = jnp.where(ci == labs[:, :, None], lp, 0.0).reshape(_GB * _T, _C)
    pm = jnp.exp(lp).reshape(_GB * _T, _C)
    ustack = jnp.concatenate([u0, u1, cs[5:6, :]], axis=0)[:, 0:_C]
    ucols = jnp.transpose(ustack)
    v = jax.lax.dot_general(
        pm, ucols, (((1,), (0,)), ((), ())),
        preferred_element_type=jnp.float32,
    )
    gv = jax.lax.dot_general(
        mlp, jnp.transpose(cs[6:7, :])[0:_C, :], (((1,), (0,)), ((), ())),
        preferred_element_type=jnp.float32,
    )
    vg = jnp.concatenate([v, gv], axis=1)
    v3 = vg.reshape(_GB, _T, 4)
    vt = jnp.transpose(v3, (0, 2, 1))
    g_ref[...] = vt[:, 3, :]
    s00 = jnp.log(vt[:, 0, :])
    s10 = jnp.log(vt[:, 1, :])
    e2 = jnp.log(vt[:, 2, :])
    s01 = e2 + c01
    s11 = e2 + c11

    lens = lens_ref[...][:, 0:1]
    tt = lax.broadcasted_iota(jnp.int32, (_GB, _T), 1)
    act = tt < lens
    a00 = jnp.where(act, s00, 0.0)
    a01 = jnp.where(act, s01, _NEG)
    a10 = jnp.where(act, s10, _NEG)
    a11 = jnp.where(act, s11, 0.0)

    n = _T
    for k in range(10):
        sh = 1 << k

        def shift(x, fill):
            pad = jnp.full((_GB, sh), fill, x.dtype)
            return jnp.concatenate([pad, x[:, : n - sh]], axis=1)

        b00 = shift(a00, 0.0)
        b01 = shift(a01, _NEG)
        b10 = shift(a10, _NEG)
        b11 = shift(a11, 0.0)
        c00 = _lse2(b00 + a00, b01 + a10)
        a00_new = jnp.maximum(c00, _NEG)
        if k < 9:
            c01_ = _lse2(b00 + a01, b01 + a11)
            c10 = _lse2(b10 + a00, b11 + a10)
            c11_ = _lse2(b10 + a01, b11 + a11)
            a01 = jnp.maximum(c01_, _NEG)
            a10 = jnp.maximum(c10, _NEG)
            a11 = jnp.maximum(c11_, _NEG)
        a00 = a00_new

    p00 = a00[:, _T - 1 : _T]
    block_den = jnp.sum(p00) + _GB * wf

    @pl.when(pl.program_id(0) == 0)
    def _():
        out_ref[0, 0] = 0.0

    out_ref[0, 0] += block_den


def _den_call(log_probs, input_lens, labels, den_scores):
    dens = jnp.zeros((1, _PAD), jnp.float32).at[0, :_A].set(den_scores)
    lens2 = jnp.broadcast_to(input_lens[:, None], (_B, 8)).astype(jnp.int32)
    out, g = pl.pallas_call(
        _den_body,
        grid=(_GRID,),
        in_specs=[
            pl.BlockSpec((8, _PAD), lambda i: (0, 0)),
            pl.BlockSpec((2 * _PAD, _PAD), lambda i: (0, 0)),
            pl.BlockSpec((1, _PAD), lambda i: (0, 0)),
            pl.BlockSpec((_GB, 8), lambda i: (i, 0)),
            pl.BlockSpec((_GB, _T), lambda i: (i, 0)),
            pl.BlockSpec((_GB, _T, _C), lambda i: (i, 0, 0)),
        ],
        out_specs=[
            pl.BlockSpec((1, 1), lambda i: (0, 0), memory_space=pltpu.SMEM),
            pl.BlockSpec((_GB, _T), lambda i: (i, 0)),
        ],
        out_shape=[
            jax.ShapeDtypeStruct((1, 1), jnp.float32),
            jax.ShapeDtypeStruct((_B, _T), jnp.float32),
        ],
        compiler_params=pltpu.CompilerParams(
            dimension_semantics=("arbitrary",)
        ),
    )(jnp.asarray(_CS), jnp.asarray(_M01), dens, lens2,
      labels.astype(jnp.int32), log_probs)
    return out[0, 0], g


_NW = 32
_BPW = _B // _NW
_NCH = 8


def _num_body(g_hbm, len_hbm, out_hbm, g_v, len_v, acc_v, sem0, sem1):
    wid = lax.axis_index("s") * 2 + lax.axis_index("c")
    sems = [sem0, sem1]
    pltpu.sync_copy(len_hbm, len_v)
    iot = lax.iota(jnp.int32, 16)
    b0 = wid * _BPW
    copies = [
        pltpu.async_copy(g_hbm.at[b0 + i], g_v.at[i], sems[i])
        for i in range(_BPW)
    ]
    acc = jnp.zeros((16,), jnp.float32)
    for i in range(_BPW):
        b = b0 + i
        copies[i].wait()
        lenb = plsc.load_gather(len_v, [jnp.full((16,), b, jnp.int32)])
        for u in range(_T // 16):
            tvec = u * 16 + iot
            val = g_v[i, pl.ds(u * 16, 16)]
            acc = acc + jnp.where(tvec < lenb, val, 0.0)
    acc_v[...] = acc
    pltpu.sync_copy(acc_v, out_hbm.at[wid])


def _num_call(g, input_lens):
    mesh = plsc.VectorSubcoreMesh(core_axis_name="c", subcore_axis_name="s")
    fn = pl.kernel(
        _num_body,
        out_type=jax.ShapeDtypeStruct((_NW, 16), jnp.float32),
        mesh=mesh,
        scratch_types=[
            pltpu.VMEM((_BPW, _T), jnp.float32),
            pltpu.VMEM((_B,), jnp.int32),
            pltpu.VMEM((16,), jnp.float32),
            pltpu.SemaphoreType.DMA,
            pltpu.SemaphoreType.DMA,
        ],
        compiler_params=pltpu.CompilerParams(needs_layout_passes=False),
    )
    parts = fn(g, input_lens.astype(jnp.int32))
    return jnp.sum(parts)


def kernel(log_probs, input_lens, labels, den_scores):
    den, g = _den_call(log_probs, input_lens, labels, den_scores)
    num = _num_call(g, input_lens)
    return num - den


# speedup vs baseline: 1.9971x; 1.9971x over previous
import functools

import numpy as np
import jax
import jax.numpy as jnp
from jax import lax
from jax.experimental import pallas as pl
from jax.experimental.pallas import tpu as pltpu
from jax.experimental.pallas import tpu_sc as plsc

_L = 40
_B, _T, _C = 64, 1024, 43
_NEG = -1e30


def _topology(num_labels):
    s = ["0 0 1", "0 1 2", "1 1 2"]
    for i in range(num_labels):
        sym = 3 + i
        s.append("0 0 %d" % sym)
        s.append("1 0 %d" % sym)
    s.append("0 2 -1")
    s.sort()
    return [tuple(int(x) for x in line.split()) for line in s]


_ARCS = _topology(_L)
_A = len(_ARCS)
_SRCA = np.array([a[0] for a in _ARCS], dtype=np.int32)
_DSTA = np.array([a[1] for a in _ARCS], dtype=np.int32)
_LABA = np.array([a[2] for a in _ARCS], dtype=np.int32)

_PAD = 128
_MS0 = np.zeros((1, _PAD), np.float32)
_MS0[0, :_A] = (_SRCA == 0)
_MS1 = np.zeros((1, _PAD), np.float32)
_MS1[0, :_A] = (_SRCA == 1)
_M0 = np.zeros((_PAD, _PAD), np.float32)
_M1 = np.zeros((_PAD, _PAD), np.float32)
for _a in range(_A):
    if _LABA[_a] >= 0 and _DSTA[_a] == 0:
        if _SRCA[_a] == 0:
            _M0[_a, _LABA[_a]] = 1.0
        else:
            _M1[_a, _LABA[_a]] = 1.0
_OH01 = np.zeros((1, _PAD), np.float32)
_OH11 = np.zeros((1, _PAD), np.float32)
_OHF = np.zeros((1, _PAD), np.float32)
for _a in range(_A):
    if _LABA[_a] < 0:
        _OHF[0, _a] = 1.0
    elif _SRCA[_a] == 0 and _DSTA[_a] == 1:
        _OH01[0, _a] = 1.0
    elif _SRCA[_a] == 1 and _DSTA[_a] == 1:
        _OH11[0, _a] = 1.0
_OHC2 = np.zeros((1, _PAD), np.float32)
_OHC2[0, 2] = 1.0

_ONESC = np.zeros((1, _PAD), np.float32)
_ONESC[0, :_C] = 1.0

_CS = np.concatenate([_MS0, _MS1, _OH01, _OH11, _OHF, _OHC2, _ONESC,
                      np.zeros((1, _PAD), np.float32)], axis=0)
_M01 = np.concatenate([_M0, _M1], axis=0)

_GB = 8
_GRID = _B // _GB


def _lse2(x, y):
    m = jnp.maximum(x, y)
    return m + jnp.log1p(jnp.exp(-jnp.abs(x - y)))


def _den_body(cs_ref, m_ref, dens_ref, lens_ref, labs_ref, lp_ref,
              out_ref, g_ref):
    d = dens_ref[...]
    cs = cs_ref[...]
    ms0 = cs[0:1, :]
    ms1 = cs[1:2, :]
    e = jnp.exp(d)
    lse0 = jnp.log(jnp.sum(e * ms0))
    lse1 = jnp.log(jnp.sum(e * ms1))
    w = d - ms0 * lse0 - ms1 * lse1
    ew = jnp.exp(w) * (ms0 + ms1)
    u0 = jnp.dot(ew, m_ref[0:_PAD, :])
    u1 = jnp.dot(ew, m_ref[_PAD:, :])
    c01 = jnp.sum(w * cs[2:3, :])
    c11 = jnp.sum(w * cs[3:4, :])
    wf = jnp.sum(w * cs[4:5, :])

    u0c = jnp.transpose(u0)[0:_C].reshape(_C, 1, 1)
    u1c = jnp.transpose(u1)[0:_C].reshape(_C, 1, 1)
    lpb = lp_ref[...]
    p = jnp.exp(lpb)
    s0 = jnp.sum(p * u0c, axis=0)
    s1 = jnp.sum(p * u1c, axis=0)
    labs = labs_ref[...]
    ci = lax.broadcasted_iota(jnp.int32, (_C, _GB, _T), 0)
    g_ref[...] = jnp.sum(jnp.where(ci == labs[None, :, :], lpb, 0.0), axis=0)
    s00 = jnp.log(s0)
    s10 = jnp.log(s1)
    e2 = lpb[2, :, :]
    s01 = e2 + c01
    s11 = e2 + c11

    lens = lens_ref[...][:, 0:1]
    tt = lax.broadcasted_iota(jnp.int32, (_GB, _T), 1)
    act = tt < lens
    a00 = jnp.where(act, s00, 0.0)
    a01 = jnp.where(act, s01, _NEG)
    a10 = jnp.where(act, s10, _NEG)
    a11 = jnp.where(act, s11, 0.0)

    n = _T
    for k in range(10):
        sh = 1 << k

        def shift(x, fill):
            pad = jnp.full((_GB, sh), fill, x.dtype)
            return jnp.concatenate([pad, x[:, : n - sh]], axis=1)

        b00 = shift(a00, 0.0)
        b01 = shift(a01, _NEG)
        b10 = shift(a10, _NEG)
        b11 = shift(a11, 0.0)
        c00 = _lse2(b00 + a00, b01 + a10)
        a00_new = jnp.maximum(c00, _NEG)
        if k < 9:
            c01_ = _lse2(b00 + a01, b01 + a11)
            c10 = _lse2(b10 + a00, b11 + a10)
            c11_ = _lse2(b10 + a01, b11 + a11)
            a01 = jnp.maximum(c01_, _NEG)
            a10 = jnp.maximum(c10, _NEG)
            a11 = jnp.maximum(c11_, _NEG)
        a00 = a00_new

    p00 = a00[:, _T - 1 : _T]
    block_den = jnp.sum(p00) + _GB * wf

    @pl.when(pl.program_id(0) == 0)
    def _():
        out_ref[0, 0] = 0.0

    out_ref[0, 0] += block_den


def _den_call(log_probs, input_lens, labels, den_scores):
    dens = jnp.zeros((1, _PAD), jnp.float32).at[0, :_A].set(den_scores)
    lens2 = jnp.broadcast_to(input_lens[:, None], (_B, 8)).astype(jnp.int32)
    out, g = pl.pallas_call(
        _den_body,
        grid=(_GRID,),
        in_specs=[
            pl.BlockSpec((8, _PAD), lambda i: (0, 0)),
            pl.BlockSpec((2 * _PAD, _PAD), lambda i: (0, 0)),
            pl.BlockSpec((1, _PAD), lambda i: (0, 0)),
            pl.BlockSpec((_GB, 8), lambda i: (i, 0)),
            pl.BlockSpec((_GB, _T), lambda i: (i, 0)),
            pl.BlockSpec((_C, _GB, _T), lambda i: (0, i, 0)),
        ],
        out_specs=[
            pl.BlockSpec((1, 1), lambda i: (0, 0), memory_space=pltpu.SMEM),
            pl.BlockSpec((_GB, _T), lambda i: (i, 0)),
        ],
        out_shape=[
            jax.ShapeDtypeStruct((1, 1), jnp.float32),
            jax.ShapeDtypeStruct((_B, _T), jnp.float32),
        ],
        compiler_params=pltpu.CompilerParams(
            dimension_semantics=("arbitrary",)
        ),
    )(jnp.asarray(_CS), jnp.asarray(_M01), dens, lens2,
      labels.astype(jnp.int32), jnp.transpose(log_probs, (2, 0, 1)))
    return out[0, 0], g


_NW = 32
_BPW = _B // _NW
_NCH = 8


def _num_body(g_hbm, len_hbm, out_hbm, g_v, len_v, acc_v, sem0, sem1):
    wid = lax.axis_index("s") * 2 + lax.axis_index("c")
    sems = [sem0, sem1]
    pltpu.sync_copy(len_hbm, len_v)
    iot = lax.iota(jnp.int32, 16)
    b0 = wid * _BPW
    copies = [
        pltpu.async_copy(g_hbm.at[b0 + i], g_v.at[i], sems[i])
        for i in range(_BPW)
    ]
    acc = jnp.zeros((16,), jnp.float32)
    for i in range(_BPW):
        b = b0 + i
        copies[i].wait()
        lenb = plsc.load_gather(len_v, [jnp.full((16,), b, jnp.int32)])
        for u in range(_T // 16):
            tvec = u * 16 + iot
            val = g_v[i, pl.ds(u * 16, 16)]
            acc = acc + jnp.where(tvec < lenb, val, 0.0)
    acc_v[...] = acc
    pltpu.sync_copy(acc_v, out_hbm.at[wid])


def _num_call(g, input_lens):
    mesh = plsc.VectorSubcoreMesh(core_axis_name="c", subcore_axis_name="s")
    fn = pl.kernel(
        _num_body,
        out_type=jax.ShapeDtypeStruct((_NW, 16), jnp.float32),
        mesh=mesh,
        scratch_types=[
            pltpu.VMEM((_BPW, _T), jnp.float32),
            pltpu.VMEM((_B,), jnp.int32),
            pltpu.VMEM((16,), jnp.float32),
            pltpu.SemaphoreType.DMA,
            pltpu.SemaphoreType.DMA,
        ],
        compiler_params=pltpu.CompilerParams(needs_layout_passes=False),
    )
    parts = fn(g, input_lens.astype(jnp.int32))
    return jnp.sum(parts)


def kernel(log_probs, input_lens, labels, den_scores):
    den, g = _den_call(log_probs, input_lens, labels, den_scores)
    num = _num_call(g, input_lens)
    return num - den
